# baseline (speedup 1.0000x reference)
import functools

import jax
import jax.numpy as jnp
from jax import lax
from jax.experimental import pallas as pl
from jax.experimental.pallas import tpu as pltpu
from jax.experimental.pallas import tpu_sc as plsc

N = 3_200_000
S = 8192
NC, NS, L = 2, 16, 16
NW = NC * NS
R = N // NW
C = 4_000
NCHUNK = R // C
G = C // L
SEG = S // NS

TC_BLK = 640_000

_GATHER_DNUMS = lax.GatherDimensionNumbers(
    offset_dims=(), collapsed_slice_dims=(0,), start_index_map=(0,))


def _shift_up(x, iota):
    idx = jnp.minimum(iota + 1, L - 1)
    return lax.gather(x, idx[:, None], _GATHER_DNUMS, slice_sizes=(1,),
                      mode=lax.GatherScatterMode.PROMISE_IN_BOUNDS)


def _rowerr_body(p_ref, t_ref, v_ref):
    d = p_ref[...] - t_ref[...]
    e = d * d
    v_ref[...] = e[0, :] + e[1, :] + e[2, :]


@jax.jit
def _rowerr(pT, tT):
    return pl.pallas_call(
        _rowerr_body,
        grid=(N // TC_BLK,),
        in_specs=[pl.BlockSpec((3, TC_BLK), lambda i: (0, i)),
                  pl.BlockSpec((3, TC_BLK), lambda i: (0, i))],
        out_specs=pl.BlockSpec((TC_BLK,), lambda i: (i,)),
        out_shape=jax.ShapeDtypeStruct((N,), jnp.float32),
        compiler_params=pltpu.CompilerParams(
            dimension_semantics=("arbitrary",)),
    )(pT, tT)


def _sc_body(v_hbm, ids_hbm, out_hbm,
             vb0, idb0, vb1, idb1,
             acc, shared, colbuf, sem0, sem1):
    c = lax.axis_index("c")
    s = lax.axis_index("s")
    wid = c * NS + s
    row0 = wid * R

    hbms = (v_hbm, ids_hbm)
    slot_bufs = ((vb0, idb0), (vb1, idb1))
    sems = (sem0, sem1)

    iota = lax.iota(jnp.int32, L)
    zeros = jnp.zeros((L,), jnp.float32)
    lane_last = iota == L - 1
    lane_not_last = iota < L - 1

    def zero_body(i, carry):
        acc[pl.ds(i * L, L)] = zeros
        return carry
    lax.fori_loop(0, S // L, zero_body, 0)

    def start_chunk(k, slot):
        base = row0 + k * C
        for hbm, buf in zip(hbms, slot_bufs[slot]):
            pltpu.async_copy(hbm.at[pl.ds(base, C)], buf, sems[slot])

    def wait_chunk(k, slot):
        base = row0 + k * C
        for hbm, buf in zip(hbms, slot_bufs[slot]):
            pltpu.make_async_copy(hbm.at[pl.ds(base, C)], buf,
                                  sems[slot]).wait()

    def compute_chunk(slot):
        vs, idss = slot_bufs[slot]
        UNROLL = 10

        def group_body(g, gcarry):
            o0 = g * (L * UNROLL)
            for u in range(UNROLL):
                o = o0 + u * L
                v = vs[pl.ds(o, L)]
                ids = idss[pl.ds(o, L)]
                cs = plsc.cumsum(v)
                ids_next = _shift_up(ids, iota)
                m_end = (ids != ids_next) | lane_last
                m_int = m_end & lane_not_last
                plsc.addupdate_scatter(acc, [ids], cs, mask=m_end)
                plsc.addupdate_scatter(acc, [ids_next], -cs, mask=m_int)
            return gcarry
        lax.fori_loop(0, G // UNROLL, group_body, 0)

    start_chunk(0, 0)
    start_chunk(1, 1)

    def pair_body(j, carry):
        k0 = 2 * j
        wait_chunk(k0, 0)
        compute_chunk(0)

        @pl.when(k0 + 2 < NCHUNK)
        def _():
            start_chunk(k0 + 2, 0)

        @pl.when(k0 + 1 < NCHUNK)
        def _():
            wait_chunk(k0 + 1, 1)
            compute_chunk(1)

            @pl.when(k0 + 3 < NCHUNK)
            def _():
                start_chunk(k0 + 3, 1)
        return carry
    lax.fori_loop(0, (NCHUNK + 1) // 2, pair_body, 0)

    pltpu.sync_copy(acc, shared.at[s])
    plsc.subcore_barrier()
    pltpu.sync_copy(shared.at[:, pl.ds(s * SEG, SEG)], colbuf)

    def col_body(i, carry):
        tot = colbuf[0, pl.ds(i * L, L)]
        for r in range(1, NS):
            tot = tot + colbuf[r, pl.ds(i * L, L)]
        acc[pl.ds(i * L, L)] = tot
        return carry
    lax.fori_loop(0, SEG // L, col_body, 0)
    pltpu.sync_copy(acc.at[pl.ds(0, SEG)], out_hbm.at[c, pl.ds(s * SEG, SEG)])


@jax.jit
def _run(v, ids):
    mesh = plsc.VectorSubcoreMesh(core_axis_name="c", subcore_axis_name="s",
                                  num_cores=NC, num_subcores=NS)
    fn = pl.kernel(
        _sc_body,
        out_type=jax.ShapeDtypeStruct((NC, S), jnp.float32),
        mesh=mesh,
        compiler_params=pltpu.CompilerParams(needs_layout_passes=False),
        scratch_types=[
            pltpu.VMEM((C,), jnp.float32),
            pltpu.VMEM((C,), jnp.int32),
            pltpu.VMEM((C,), jnp.float32),
            pltpu.VMEM((C,), jnp.int32),
            pltpu.VMEM((S,), jnp.float32),
            pltpu.VMEM_SHARED((NS, S), jnp.float32),
            pltpu.VMEM((NS, SEG), jnp.float32),
            pltpu.SemaphoreType.DMA,
            pltpu.SemaphoreType.DMA,
        ],
    )
    return fn(v, ids)


def kernel(pred_eps, true_eps, segment_ids):
    ids = segment_ids.astype(jnp.int32)
    v = _rowerr(jnp.swapaxes(pred_eps, 0, 1), jnp.swapaxes(true_eps, 0, 1))
    parts = _run(v, ids)
    return parts[0] + parts[1]


# speedup vs baseline: 1.5301x; 1.5301x over previous
import functools

import jax
import jax.numpy as jnp
from jax import lax
from jax.experimental import pallas as pl
from jax.experimental.pallas import tpu as pltpu
from jax.experimental.pallas import tpu_sc as plsc

N = 3_200_000
S = 8192
NC, NS, L = 2, 16, 16
NW = NC * NS
R = N // NW
C = 4_000
NCHUNK = R // C
G = C // L
SEG = S // NS

TC_BLK = 640_000

_GATHER_DNUMS = lax.GatherDimensionNumbers(
    offset_dims=(), collapsed_slice_dims=(0,), start_index_map=(0,))


def _shift_up(x, iota):
    idx = jnp.minimum(iota + 1, L - 1)
    return lax.gather(x, idx[:, None], _GATHER_DNUMS, slice_sizes=(1,),
                      mode=lax.GatherScatterMode.PROMISE_IN_BOUNDS)


def _rowerr_body(p_ref, t_ref, v_ref):
    d = p_ref[...] - t_ref[...]
    e = d * d
    v_ref[...] = e[0, :] + e[1, :] + e[2, :]


@jax.jit
def _rowerr(pT, tT):
    return pl.pallas_call(
        _rowerr_body,
        grid=(N // TC_BLK,),
        in_specs=[pl.BlockSpec((3, TC_BLK), lambda i: (0, i)),
                  pl.BlockSpec((3, TC_BLK), lambda i: (0, i))],
        out_specs=pl.BlockSpec((TC_BLK,), lambda i: (i,)),
        out_shape=jax.ShapeDtypeStruct((N,), jnp.float32),
        compiler_params=pltpu.CompilerParams(
            dimension_semantics=("arbitrary",)),
    )(pT, tT)


def _sc_body(v_hbm, ids_hbm, out_hbm,
             vb0, idb0, vb1, idb1,
             acc, shared, colbuf, sem0, sem1):
    c = lax.axis_index("c")
    s = lax.axis_index("s")
    wid = c * NS + s
    row0 = wid * R

    hbms = (v_hbm, ids_hbm)
    slot_bufs = ((vb0, idb0), (vb1, idb1))
    sems = (sem0, sem1)

    iota = lax.iota(jnp.int32, L)
    zeros = jnp.zeros((L,), jnp.float32)
    lane_last = iota == L - 1
    lane_not_last = iota < L - 1

    def zero_body(i, carry):
        acc[pl.ds(i * L, L)] = zeros
        return carry
    lax.fori_loop(0, S // L, zero_body, 0)

    def start_chunk(k, slot):
        base = row0 + k * C
        for hbm, buf in zip(hbms, slot_bufs[slot]):
            pltpu.async_copy(hbm.at[pl.ds(base, C)], buf, sems[slot])

    def wait_chunk(k, slot):
        base = row0 + k * C
        for hbm, buf in zip(hbms, slot_bufs[slot]):
            pltpu.make_async_copy(hbm.at[pl.ds(base, C)], buf,
                                  sems[slot]).wait()

    def compute_chunk(slot):
        vs, idss = slot_bufs[slot]
        UNROLL = 5

        def group_body(g, gcarry):
            o0 = g * (L * UNROLL)
            flushes = []
            for u in range(UNROLL):
                o = o0 + u * L
                v = vs[pl.ds(o, L)]
                ids = idss[pl.ds(o, L)]
                cs = plsc.cumsum(v)
                ids_next = _shift_up(ids, iota)
                m_end = (ids != ids_next) | lane_last
                m_int = m_end & lane_not_last
                flushes.append((ids, cs, m_end, ids_next, m_int))
            for ids, cs, m_end, ids_next, m_int in flushes:
                plsc.addupdate_scatter(acc, [ids], cs, mask=m_end)
                plsc.addupdate_scatter(acc, [ids_next], -cs, mask=m_int)
            return gcarry
        lax.fori_loop(0, G // UNROLL, group_body, 0)

    start_chunk(0, 0)
    start_chunk(1, 1)

    def pair_body(j, carry):
        k0 = 2 * j
        wait_chunk(k0, 0)
        compute_chunk(0)

        @pl.when(k0 + 2 < NCHUNK)
        def _():
            start_chunk(k0 + 2, 0)

        @pl.when(k0 + 1 < NCHUNK)
        def _():
            wait_chunk(k0 + 1, 1)
            compute_chunk(1)

            @pl.when(k0 + 3 < NCHUNK)
            def _():
                start_chunk(k0 + 3, 1)
        return carry
    lax.fori_loop(0, (NCHUNK + 1) // 2, pair_body, 0)

    pltpu.sync_copy(acc, shared.at[s])
    plsc.subcore_barrier()
    pltpu.sync_copy(shared.at[:, pl.ds(s * SEG, SEG)], colbuf)

    def col_body(i, carry):
        tot = colbuf[0, pl.ds(i * L, L)]
        for r in range(1, NS):
            tot = tot + colbuf[r, pl.ds(i * L, L)]
        acc[pl.ds(i * L, L)] = tot
        return carry
    lax.fori_loop(0, SEG // L, col_body, 0)
    pltpu.sync_copy(acc.at[pl.ds(0, SEG)], out_hbm.at[c, pl.ds(s * SEG, SEG)])


@jax.jit
def _run(v, ids):
    mesh = plsc.VectorSubcoreMesh(core_axis_name="c", subcore_axis_name="s",
                                  num_cores=NC, num_subcores=NS)
    fn = pl.kernel(
        _sc_body,
        out_type=jax.ShapeDtypeStruct((NC, S), jnp.float32),
        mesh=mesh,
        compiler_params=pltpu.CompilerParams(needs_layout_passes=False),
        scratch_types=[
            pltpu.VMEM((C,), jnp.float32),
            pltpu.VMEM((C,), jnp.int32),
            pltpu.VMEM((C,), jnp.float32),
            pltpu.VMEM((C,), jnp.int32),
            pltpu.VMEM((S,), jnp.float32),
            pltpu.VMEM_SHARED((NS, S), jnp.float32),
            pltpu.VMEM((NS, SEG), jnp.float32),
            pltpu.SemaphoreType.DMA,
            pltpu.SemaphoreType.DMA,
        ],
    )
    return fn(v, ids)


def kernel(pred_eps, true_eps, segment_ids):
    ids = segment_ids.astype(jnp.int32)
    v = _rowerr(jnp.swapaxes(pred_eps, 0, 1), jnp.swapaxes(true_eps, 0, 1))
    parts = _run(v, ids)
    return parts[0] + parts[1]
